# Initial kernel scaffold; baseline (speedup 1.0000x reference)
import functools

import jax
import jax.numpy as jnp
from jax import lax
from jax.experimental import pallas as pl
from jax.experimental.pallas import tpu as pltpu
from jax.experimental.pallas import tpu_sc as plsc

N = 10000
NE = 320000
D = 128
L = 16
NC = 2
NS = 16
NW = NC * NS
K = 128
EPT = 10240
CCH = EPT // K
EPAD = NW * EPT - NE
NP = 10240
SL = NP // NS
RB = 400

_mesh = plsc.VectorSubcoreMesh(
    core_axis_name="c", subcore_axis_name="s", num_cores=NC, num_subcores=NS)


@functools.partial(
    pl.kernel,
    out_type=jax.ShapeDtypeStruct((NC, NP, 1), jnp.float32),
    mesh=_mesh,
    scratch_types=[
        pltpu.VMEM_SHARED((NP,), jnp.float32),
        pltpu.VMEM((CCH, K), jnp.int32),
        pltpu.VMEM((K,), jnp.float32),
    ],
)
def _deg_kernel(dst_hbm, zeros_hbm, degp_hbm, deg_sp, idx_v, ones_v):
    c = lax.axis_index("c")
    s = lax.axis_index("s")
    g = c * NS + s
    pltpu.sync_copy(dst_hbm.at[g], idx_v)
    for j in range(K // L):
        ones_v[pl.ds(j * L, L)] = jnp.ones((L,), jnp.float32)

    @pl.when(s == 0)
    def _():
        pltpu.sync_copy(zeros_hbm, deg_sp)

    plsc.subcore_barrier()

    def body(i, carry):
        pltpu.sync_copy(ones_v, deg_sp.at[idx_v.at[i]], add=True)
        return carry

    lax.fori_loop(0, CCH, body, 0)
    plsc.subcore_barrier()
    pltpu.sync_copy(deg_sp.at[pl.ds(s * SL, SL)],
                    degp_hbm.at[c, pl.ds(s * SL, SL), 0])


@functools.partial(
    pl.kernel,
    out_type=jax.ShapeDtypeStruct((NC, NP, D), jnp.float32),
    mesh=_mesh,
    scratch_types=[
        pltpu.VMEM_SHARED((NP, D), jnp.float32),
        pltpu.VMEM((CCH, K), jnp.int32),
        pltpu.VMEM((CCH, K), jnp.int32),
        pltpu.VMEM((K, D), jnp.float32),
        pltpu.VMEM((K, D), jnp.float32),
        pltpu.SemaphoreType.DMA,
        pltpu.SemaphoreType.DMA,
    ],
)
def _scatter_kernel(hp_hbm, src_hbm, dst_hbm, p_hbm,
                    acc, sidx, didx, rows0, rows1, gsem0, gsem1):
    c = lax.axis_index("c")
    s = lax.axis_index("s")
    g = c * NS + s
    pltpu.sync_copy(src_hbm.at[g], sidx)
    pltpu.sync_copy(dst_hbm.at[g], didx)

    @pl.when(s < NS - 1)
    def _():
        pltpu.sync_copy(hp_hbm.at[pl.ds(s * SL, SL)],
                        acc.at[pl.ds(s * SL, SL)])

    @pl.when(s == NS - 1)
    def _():
        pltpu.sync_copy(hp_hbm.at[pl.ds((NS - 1) * SL, N - (NS - 1) * SL)],
                        acc.at[pl.ds((NS - 1) * SL, N - (NS - 1) * SL)])

    pltpu.async_copy(hp_hbm.at[sidx.at[0]], rows0, gsem0)
    pltpu.async_copy(hp_hbm.at[sidx.at[1]], rows1, gsem1)

    plsc.subcore_barrier()

    def body(k, carry):
        i = 2 * k
        pltpu.make_async_copy(hp_hbm.at[sidx.at[0]], rows0, gsem0).wait()
        pltpu.sync_copy(rows0, acc.at[didx.at[i]], add=True)

        @pl.when(i + 2 < CCH)
        def _():
            pltpu.async_copy(hp_hbm.at[sidx.at[i + 2]], rows0, gsem0)

        pltpu.make_async_copy(hp_hbm.at[sidx.at[1]], rows1, gsem1).wait()
        pltpu.sync_copy(rows1, acc.at[didx.at[i + 1]], add=True)

        @pl.when(i + 3 < CCH)
        def _():
            pltpu.async_copy(hp_hbm.at[sidx.at[i + 3]], rows1, gsem1)

        return carry

    lax.fori_loop(0, CCH // 2, body, 0)
    plsc.subcore_barrier()
    pltpu.sync_copy(acc.at[pl.ds(s * SL, SL)],
                    p_hbm.at[c, pl.ds(s * SL, SL), :])


def _h_body(dp0_ref, dp1_ref, x_ref, w_ref, b_ref, o_ref):
    deg = dp0_ref[0, :, 0] + dp1_ref[0, :, 0] + 1.0
    dinv = lax.rsqrt(deg)
    h = jnp.dot(x_ref[...], w_ref[...],
                preferred_element_type=jnp.float32) + b_ref[0, :][None, :]
    o_ref[...] = h * dinv[:, None]


def _out_body(dp0_ref, dp1_ref, p0_ref, p1_ref, hp_ref, o_ref):
    deg = dp0_ref[0, :, 0] + dp1_ref[0, :, 0] + 1.0
    dinv = lax.rsqrt(deg)
    acc = p0_ref[0] + p1_ref[0] - hp_ref[...]
    o_ref[...] = acc * dinv[:, None]


def kernel(V, E, X, W, b):
    src = E[0].astype(jnp.int32)
    dst = E[1].astype(jnp.int32)
    srcp = jnp.concatenate(
        [src, jnp.zeros((EPAD,), jnp.int32)]).reshape(NW, CCH, K)
    dstp = jnp.concatenate(
        [dst, jnp.full((EPAD,), N, jnp.int32)]).reshape(NW, CCH, K)
    zeros_np = jnp.zeros((NP,), jnp.float32)
    b2 = b.reshape(1, D).astype(jnp.float32)

    dp = _deg_kernel(dstp, zeros_np)

    grid = (N // RB,)
    hp = pl.pallas_call(
        _h_body,
        grid=grid,
        in_specs=[
            pl.BlockSpec((1, RB, 1), lambda i: (0, i, 0)),
            pl.BlockSpec((1, RB, 1), lambda i: (1, i, 0)),
            pl.BlockSpec((RB, D), lambda i: (i, 0)),
            pl.BlockSpec((D, D), lambda i: (0, 0)),
            pl.BlockSpec((1, D), lambda i: (0, 0)),
        ],
        out_specs=pl.BlockSpec((RB, D), lambda i: (i, 0)),
        out_shape=jax.ShapeDtypeStruct((N, D), jnp.float32),
    )(dp, dp, X.astype(jnp.float32), W.astype(jnp.float32), b2)

    p = _scatter_kernel(hp, srcp, dstp)

    out = pl.pallas_call(
        _out_body,
        grid=grid,
        in_specs=[
            pl.BlockSpec((1, RB, 1), lambda i: (0, i, 0)),
            pl.BlockSpec((1, RB, 1), lambda i: (1, i, 0)),
            pl.BlockSpec((1, RB, D), lambda i: (0, i, 0)),
            pl.BlockSpec((1, RB, D), lambda i: (1, i, 0)),
            pl.BlockSpec((RB, D), lambda i: (i, 0)),
        ],
        out_specs=pl.BlockSpec((RB, D), lambda i: (i, 0)),
        out_shape=jax.ShapeDtypeStruct((N, D), jnp.float32),
    )(dp, dp, p, p, hp)
    return out


# speedup vs baseline: 14.8066x; 14.8066x over previous
import functools

import jax
import jax.numpy as jnp
from jax import lax
from jax.experimental import pallas as pl
from jax.experimental.pallas import tpu as pltpu
from jax.experimental.pallas import tpu_sc as plsc

N = 10000
NE = 320000
D = 128
L = 16
NC = 2
NS = 16
NW = NC * NS
K = 128
EPT = 10240
CCH = EPT // K
EPAD = NW * EPT - NE
NP = 10240
SL = NP // NS
RB = 400
GB = 16
NG = CCH // GB

_mesh = plsc.VectorSubcoreMesh(
    core_axis_name="c", subcore_axis_name="s", num_cores=NC, num_subcores=NS)


@functools.partial(
    pl.kernel,
    out_type=[jax.ShapeDtypeStruct((NP,), jnp.float32),
              jax.ShapeDtypeStruct((NP,), jnp.float32)],
    mesh=_mesh,
    scratch_types=[
        pltpu.VMEM_SHARED((NP,), jnp.float32),
        pltpu.VMEM((CCH, K), jnp.int32),
        pltpu.VMEM((K,), jnp.float32),
    ],
)
def _deg_kernel(dst_hbm, zeros_hbm, degp0_hbm, degp1_hbm, deg_sp, idx_v,
                ones_v):
    c = lax.axis_index("c")
    s = lax.axis_index("s")
    g = c * NS + s
    pltpu.sync_copy(dst_hbm.at[g], idx_v)
    for j in range(K // L):
        ones_v[pl.ds(j * L, L)] = jnp.ones((L,), jnp.float32)

    @pl.when(s == 0)
    def _():
        pltpu.sync_copy(zeros_hbm, deg_sp)

    plsc.subcore_barrier()

    def body(i, carry):
        pltpu.sync_copy(ones_v, deg_sp.at[idx_v.at[i]], add=True)
        return carry

    lax.fori_loop(0, CCH, body, 0)
    plsc.subcore_barrier()
    sl = pl.ds(s * SL, SL)

    @pl.when(c == 0)
    def _():
        pltpu.sync_copy(deg_sp.at[sl], degp0_hbm.at[sl])

    @pl.when(c == 1)
    def _():
        pltpu.sync_copy(deg_sp.at[sl], degp1_hbm.at[sl])


@functools.partial(
    pl.kernel,
    out_type=jax.ShapeDtypeStruct((NC, NP, D), jnp.float32),
    mesh=_mesh,
    scratch_types=[
        pltpu.VMEM_SHARED((NP, D), jnp.float32),
        pltpu.VMEM((GB, K), jnp.int32),
        pltpu.VMEM((GB, K), jnp.int32),
        pltpu.VMEM((K, D), jnp.float32),
        pltpu.VMEM((K, D), jnp.float32),
        pltpu.SemaphoreType.DMA,
        pltpu.SemaphoreType.DMA,
    ],
)
def _scatter_kernel(hp_hbm, src_hbm, dst_hbm, p_hbm,
                    acc, sidx, didx, rows0, rows1, gsem0, gsem1):
    c = lax.axis_index("c")
    s = lax.axis_index("s")
    g = c * NS + s

    def wait0():
        pltpu.make_async_copy(hp_hbm.at[sidx.at[0]], rows0, gsem0).wait()

    def wait1():
        pltpu.make_async_copy(hp_hbm.at[sidx.at[1]], rows1, gsem1).wait()

    @pl.when(s < NS - 1)
    def _():
        pltpu.sync_copy(hp_hbm.at[pl.ds(s * SL, SL)],
                        acc.at[pl.ds(s * SL, SL)])

    @pl.when(s == NS - 1)
    def _():
        pltpu.sync_copy(hp_hbm.at[pl.ds((NS - 1) * SL, N - (NS - 1) * SL)],
                        acc.at[pl.ds((NS - 1) * SL, N - (NS - 1) * SL)])

    pltpu.sync_copy(src_hbm.at[g, pl.ds(0, GB)], sidx)
    pltpu.sync_copy(dst_hbm.at[g, pl.ds(0, GB)], didx)
    pltpu.async_copy(hp_hbm.at[sidx.at[0]], rows0, gsem0)
    pltpu.async_copy(hp_hbm.at[sidx.at[1]], rows1, gsem1)

    plsc.subcore_barrier()

    def group_body(gg, carry):
        def pair_body(k, cc):
            j = 2 * k
            wait0()
            pltpu.sync_copy(rows0, acc.at[didx.at[j]], add=True)

            @pl.when(j + 2 < GB)
            def _():
                pltpu.async_copy(hp_hbm.at[sidx.at[j + 2]], rows0, gsem0)

            wait1()
            pltpu.sync_copy(rows1, acc.at[didx.at[j + 1]], add=True)

            @pl.when(j + 3 < GB)
            def _():
                pltpu.async_copy(hp_hbm.at[sidx.at[j + 3]], rows1, gsem1)

            return cc

        lax.fori_loop(0, GB // 2, pair_body, 0)

        @pl.when(gg + 1 < NG)
        def _():
            pltpu.sync_copy(src_hbm.at[g, pl.ds((gg + 1) * GB, GB)], sidx)
            pltpu.sync_copy(dst_hbm.at[g, pl.ds((gg + 1) * GB, GB)], didx)
            pltpu.async_copy(hp_hbm.at[sidx.at[0]], rows0, gsem0)
            pltpu.async_copy(hp_hbm.at[sidx.at[1]], rows1, gsem1)

        return carry

    lax.fori_loop(0, NG, group_body, 0)
    plsc.subcore_barrier()
    pltpu.sync_copy(acc.at[pl.ds(s * SL, SL)],
                    p_hbm.at[c, pl.ds(s * SL, SL), :])


def _h_body(dp0_ref, dp1_ref, x_ref, w_ref, b_ref, o_ref):
    deg = dp0_ref[...] + dp1_ref[...] + 1.0
    dinv = lax.rsqrt(deg)
    h = jnp.dot(x_ref[...], w_ref[...],
                preferred_element_type=jnp.float32) + b_ref[0, :][None, :]
    o_ref[...] = h * dinv


def _out_body(dp0_ref, dp1_ref, p0_ref, p1_ref, hp_ref, o_ref):
    deg = dp0_ref[...] + dp1_ref[...] + 1.0
    dinv = lax.rsqrt(deg)
    acc = p0_ref[0] + p1_ref[0] - hp_ref[...]
    o_ref[...] = acc * dinv


def kernel(V, E, X, W, b):
    src = E[0].astype(jnp.int32)
    dst = E[1].astype(jnp.int32)
    srcp = jnp.concatenate(
        [src, jnp.zeros((EPAD,), jnp.int32)]).reshape(NW, CCH, K)
    dstp = jnp.concatenate(
        [dst, jnp.full((EPAD,), N, jnp.int32)]).reshape(NW, CCH, K)
    zeros_np = jnp.zeros((NP,), jnp.float32)
    b2 = b.reshape(1, D).astype(jnp.float32)

    dp0, dp1 = _deg_kernel(dstp, zeros_np)
    dc0 = dp0[:N, None]
    dc1 = dp1[:N, None]

    grid = (N // RB,)
    hp = pl.pallas_call(
        _h_body,
        grid=grid,
        in_specs=[
            pl.BlockSpec((RB, 1), lambda i: (i, 0)),
            pl.BlockSpec((RB, 1), lambda i: (i, 0)),
            pl.BlockSpec((RB, D), lambda i: (i, 0)),
            pl.BlockSpec((D, D), lambda i: (0, 0)),
            pl.BlockSpec((1, D), lambda i: (0, 0)),
        ],
        out_specs=pl.BlockSpec((RB, D), lambda i: (i, 0)),
        out_shape=jax.ShapeDtypeStruct((N, D), jnp.float32),
    )(dc0, dc1, X.astype(jnp.float32), W.astype(jnp.float32), b2)

    p = _scatter_kernel(hp, srcp, dstp)

    out = pl.pallas_call(
        _out_body,
        grid=grid,
        in_specs=[
            pl.BlockSpec((RB, 1), lambda i: (i, 0)),
            pl.BlockSpec((RB, 1), lambda i: (i, 0)),
            pl.BlockSpec((1, RB, D), lambda i: (0, i, 0)),
            pl.BlockSpec((1, RB, D), lambda i: (1, i, 0)),
            pl.BlockSpec((RB, D), lambda i: (i, 0)),
        ],
        out_specs=pl.BlockSpec((RB, D), lambda i: (i, 0)),
        out_shape=jax.ShapeDtypeStruct((N, D), jnp.float32),
    )(dc0, dc1, p, p, hp)
    return out


# speedup vs baseline: 14.8515x; 1.0030x over previous
import functools

import jax
import jax.numpy as jnp
from jax import lax
from jax.experimental import pallas as pl
from jax.experimental.pallas import tpu as pltpu
from jax.experimental.pallas import tpu_sc as plsc

N = 10000
NE = 320000
D = 128
L = 16
NC = 2
NS = 16
NW = NC * NS
K = 128
EPT = 10240
CCH = EPT // K
EPAD = NW * EPT - NE
NP = 10240
SL = NP // NS
RB = 400
GB = 16
NG = CCH // GB

_mesh = plsc.VectorSubcoreMesh(
    core_axis_name="c", subcore_axis_name="s", num_cores=NC, num_subcores=NS)


@functools.partial(
    pl.kernel,
    out_type=[jax.ShapeDtypeStruct((NP,), jnp.float32),
              jax.ShapeDtypeStruct((NP,), jnp.float32)],
    mesh=_mesh,
    scratch_types=[
        pltpu.VMEM_SHARED((NP,), jnp.float32),
        pltpu.VMEM((CCH, K), jnp.int32),
        pltpu.VMEM((K,), jnp.float32),
    ],
)
def _deg_kernel(dst_hbm, zeros_hbm, degp0_hbm, degp1_hbm, deg_sp, idx_v,
                ones_v):
    c = lax.axis_index("c")
    s = lax.axis_index("s")
    g = c * NS + s
    pltpu.sync_copy(dst_hbm.at[g], idx_v)
    for j in range(K // L):
        ones_v[pl.ds(j * L, L)] = jnp.ones((L,), jnp.float32)

    @pl.when(s == 0)
    def _():
        pltpu.sync_copy(zeros_hbm, deg_sp)

    plsc.subcore_barrier()

    def body(i, carry):
        pltpu.sync_copy(ones_v, deg_sp.at[idx_v.at[i]], add=True)
        return carry

    lax.fori_loop(0, CCH, body, 0)
    plsc.subcore_barrier()
    sl = pl.ds(s * SL, SL)

    @pl.when(c == 0)
    def _():
        pltpu.sync_copy(deg_sp.at[sl], degp0_hbm.at[sl])

    @pl.when(c == 1)
    def _():
        pltpu.sync_copy(deg_sp.at[sl], degp1_hbm.at[sl])


@functools.partial(
    pl.kernel,
    out_type=jax.ShapeDtypeStruct((NC, NP, D), jnp.float32),
    mesh=_mesh,
    scratch_types=[
        pltpu.VMEM_SHARED((NP, D), jnp.float32),
        pltpu.VMEM((GB, K), jnp.int32),
        pltpu.VMEM((GB, K), jnp.int32),
        pltpu.VMEM((K, D), jnp.float32),
        pltpu.VMEM((K, D), jnp.float32),
        pltpu.SemaphoreType.DMA,
        pltpu.SemaphoreType.DMA,
    ],
)
def _scatter_kernel(hp_hbm, src_hbm, dst_hbm, p_hbm,
                    acc, sidx, didx, rows0, rows1, gsem0, gsem1):
    c = lax.axis_index("c")
    s = lax.axis_index("s")
    g = c * NS + s

    def wait0():
        pltpu.make_async_copy(hp_hbm.at[sidx.at[0]], rows0, gsem0).wait()

    def wait1():
        pltpu.make_async_copy(hp_hbm.at[sidx.at[1]], rows1, gsem1).wait()

    @pl.when(s < NS - 1)
    def _():
        pltpu.sync_copy(hp_hbm.at[pl.ds(s * SL, SL)],
                        acc.at[pl.ds(s * SL, SL)])

    @pl.when(s == NS - 1)
    def _():
        pltpu.sync_copy(hp_hbm.at[pl.ds((NS - 1) * SL, N - (NS - 1) * SL)],
                        acc.at[pl.ds((NS - 1) * SL, N - (NS - 1) * SL)])

    pltpu.sync_copy(src_hbm.at[g, pl.ds(0, GB)], sidx)
    pltpu.sync_copy(dst_hbm.at[g, pl.ds(0, GB)], didx)
    pltpu.async_copy(hp_hbm.at[sidx.at[0]], rows0, gsem0)
    pltpu.async_copy(hp_hbm.at[sidx.at[1]], rows1, gsem1)

    plsc.subcore_barrier()

    def group_body(gg, carry):
        def pair_body(k, cc):
            j = 2 * k
            wait0()
            pltpu.sync_copy(rows0, acc.at[didx.at[j]], add=True)

            @pl.when(j + 2 < GB)
            def _():
                pltpu.async_copy(hp_hbm.at[sidx.at[j + 2]], rows0, gsem0)

            wait1()
            pltpu.sync_copy(rows1, acc.at[didx.at[j + 1]], add=True)

            @pl.when(j + 3 < GB)
            def _():
                pltpu.async_copy(hp_hbm.at[sidx.at[j + 3]], rows1, gsem1)

            return cc

        lax.fori_loop(0, GB // 2, pair_body, 0)

        @pl.when(gg + 1 < NG)
        def _():
            pltpu.sync_copy(src_hbm.at[g, pl.ds((gg + 1) * GB, GB)], sidx)
            pltpu.sync_copy(dst_hbm.at[g, pl.ds((gg + 1) * GB, GB)], didx)
            pltpu.async_copy(hp_hbm.at[sidx.at[0]], rows0, gsem0)
            pltpu.async_copy(hp_hbm.at[sidx.at[1]], rows1, gsem1)

        return carry

    lax.fori_loop(0, NG, group_body, 0)
    plsc.subcore_barrier()
    pltpu.sync_copy(acc.at[pl.ds(s * SL, SL)],
                    p_hbm.at[c, pl.ds(s * SL, SL), :])


def _h_body(dp0_ref, dp1_ref, x_ref, w_ref, b_ref, o_ref):
    deg = dp0_ref[...] + dp1_ref[...] + 1.0
    dinv = lax.rsqrt(deg)
    h = jnp.dot(x_ref[...], w_ref[...],
                preferred_element_type=jnp.float32) + b_ref[0, :][None, :]
    o_ref[...] = h * dinv


def _out_body(dp0_ref, dp1_ref, p0_ref, p1_ref, hp_ref, o_ref):
    deg = dp0_ref[...] + dp1_ref[...] + 1.0
    dinv = lax.rsqrt(deg)
    acc = p0_ref[0] + p1_ref[0] - hp_ref[...]
    o_ref[...] = acc * dinv


def kernel(V, E, X, W, b):
    src = E[0].astype(jnp.int32)
    dst = E[1].astype(jnp.int32)
    srcp = jnp.concatenate(
        [src, jnp.zeros((EPAD,), jnp.int32)]).reshape(NW, CCH, K)
    pad_dst = N + (jnp.arange(EPAD, dtype=jnp.int32) % (NP - N))
    dstp = jnp.concatenate([dst, pad_dst]).reshape(NW, CCH, K)
    zeros_np = jnp.zeros((NP,), jnp.float32)
    b2 = b.reshape(1, D).astype(jnp.float32)

    dp0, dp1 = _deg_kernel(dstp, zeros_np)
    dc0 = dp0[:N, None]
    dc1 = dp1[:N, None]

    grid = (N // RB,)
    hp = pl.pallas_call(
        _h_body,
        grid=grid,
        in_specs=[
            pl.BlockSpec((RB, 1), lambda i: (i, 0)),
            pl.BlockSpec((RB, 1), lambda i: (i, 0)),
            pl.BlockSpec((RB, D), lambda i: (i, 0)),
            pl.BlockSpec((D, D), lambda i: (0, 0)),
            pl.BlockSpec((1, D), lambda i: (0, 0)),
        ],
        out_specs=pl.BlockSpec((RB, D), lambda i: (i, 0)),
        out_shape=jax.ShapeDtypeStruct((N, D), jnp.float32),
    )(dc0, dc1, X.astype(jnp.float32), W.astype(jnp.float32), b2)

    p = _scatter_kernel(hp, srcp, dstp)

    out = pl.pallas_call(
        _out_body,
        grid=grid,
        in_specs=[
            pl.BlockSpec((RB, 1), lambda i: (i, 0)),
            pl.BlockSpec((RB, 1), lambda i: (i, 0)),
            pl.BlockSpec((1, RB, D), lambda i: (0, i, 0)),
            pl.BlockSpec((1, RB, D), lambda i: (1, i, 0)),
            pl.BlockSpec((RB, D), lambda i: (i, 0)),
        ],
        out_specs=pl.BlockSpec((RB, D), lambda i: (i, 0)),
        out_shape=jax.ShapeDtypeStruct((N, D), jnp.float32),
    )(dc0, dc1, p, p, hp)
    return out
